# Initial kernel scaffold; baseline (speedup 1.0000x reference)
import jax
import jax.numpy as jnp
from jax import lax
from jax.experimental import pallas as pl
from jax.experimental.pallas import tpu as pltpu
from jax.experimental.pallas import tpu_sc as plsc

N = 10000
E = 320000
D = 128
DH = 64
STEPS = 5
NCORE = 2
NTILE = 16
NP = 10240
RPT = NP // NTILE
RBLK = 128
NRB = RPT // RBLK
EPT = E // NTILE
CK = 128
NCH = -(-EPT // CK)
EPT_PAD = NCH * CK
LANES = 16
FB = DH // LANES


def _sc_body(xh, srcp, dstp, y, hbuf,
             src_v, dst_v, rowbuf, xv, hv, aggv, zv,
             deg_v, degc_v, tmp_v, g_v, degslab, agg_sp, sem):
    c = lax.axis_index("c")
    t = lax.axis_index("s")
    rows_t = t * RPT
    f32 = jnp.float32
    ones16 = jnp.ones((LANES,), f32)
    zeros16 = jnp.zeros((LANES,), f32)

    pltpu.sync_copy(srcp.at[t], src_v)
    pltpu.sync_copy(dstp.at[t], dst_v)

    def body_zero_z(i, carry):
        for f in range(FB):
            zv[i, pl.ds(f * LANES, LANES)] = zeros16
        return carry
    lax.fori_loop(0, RBLK, body_zero_z, 0)

    def body_zero_deg(i, carry):
        deg_v[pl.ds(i * LANES, LANES)] = zeros16
        return carry
    lax.fori_loop(0, NP // LANES, body_zero_deg, 0)

    def body_hist(j, carry):
        for k in range(CK // LANES):
            idx = dst_v[j, pl.ds(k * LANES, LANES)]
            plsc.addupdate_scatter(deg_v, [idx], ones16)
        return carry
    lax.fori_loop(0, NCH, body_hist, 0)

    pltpu.sync_copy(deg_v, degslab.at[t])
    plsc.subcore_barrier()

    pltpu.sync_copy(degslab.at[0, pl.ds(rows_t, RPT)], degc_v)
    for u in range(1, NTILE):
        pltpu.sync_copy(degslab.at[u, pl.ds(rows_t, RPT)], tmp_v)

        def body_acc(k, carry):
            sl = pl.ds(k * LANES, LANES)
            degc_v[sl] = degc_v[sl] + tmp_v[sl]
            return carry
        lax.fori_loop(0, RPT // LANES, body_acc, 0)

    def body_g(k, carry):
        sl = pl.ds(k * LANES, LANES)
        d = degc_v[sl]
        i = plsc.bitcast(d, jnp.int32)
        i = jnp.int32(0x5F3759DF) - lax.shift_right_logical(i, 1)
        yb = plsc.bitcast(i, f32)
        hd = 0.5 * d
        for _ in range(4):
            yb = yb * (1.5 - hd * yb * yb)
        g_v[sl] = jnp.where(d > 0.5, yb, jnp.zeros_like(yb))
        return carry
    lax.fori_loop(0, RPT // LANES, body_g, 0)

    def _bg(r_local):
        return plsc.load_gather(g_v, [jnp.full((LANES,), r_local, jnp.int32)])

    for rb in range(NRB):
        rows0 = rows_t + rb * RBLK
        pltpu.sync_copy(xh.at[c, pl.ds(rows0, RBLK)], xv)

        def body_hi(r, carry, rb=rb):
            bg = _bg(rb * RBLK + r)
            for f in range(FB):
                sl = pl.ds(f * LANES, LANES)
                hv[r, sl] = bg * xv[r, sl]
            return carry
        lax.fori_loop(0, RBLK, body_hi, 0)
        pltpu.sync_copy(hv, hbuf.at[c, pl.ds(rows0, RBLK)])
        pltpu.sync_copy(zv, agg_sp.at[pl.ds(rows0, RBLK)])

    def _scatter_phase():
        def body(j, carry):
            cp = pltpu.async_copy(hbuf.at[c].at[src_v.at[j]], rowbuf.at[0], sem)
            cp.wait()
            pltpu.sync_copy(rowbuf.at[0], agg_sp.at[dst_v.at[j]], add=True)
            return carry
        lax.fori_loop(0, NCH, body, 0)

    def _phase_e(final):
        for rb in range(NRB):
            rows0 = rows_t + rb * RBLK
            pltpu.sync_copy(agg_sp.at[pl.ds(rows0, RBLK)], aggv)
            if not final:
                pltpu.sync_copy(zv, agg_sp.at[pl.ds(rows0, RBLK)])
            pltpu.sync_copy(xh.at[c, pl.ds(rows0, RBLK)], xv)

            def body(r, carry, rb=rb):
                bg = _bg(rb * RBLK + r)
                for f in range(FB):
                    sl = pl.ds(f * LANES, LANES)
                    v = bg * aggv[r, sl] + xv[r, sl]
                    if final:
                        hv[r, sl] = 0.5 * v
                    else:
                        hv[r, sl] = (0.5 * bg) * v
                return carry
            lax.fori_loop(0, RBLK, body, 0)
            dstref = y if final else hbuf
            pltpu.sync_copy(hv, dstref.at[c, pl.ds(rows0, RBLK)])

    plsc.subcore_barrier()

    def step_body(k, carry):
        _scatter_phase()
        plsc.subcore_barrier()
        _phase_e(False)
        plsc.subcore_barrier()
        return carry
    lax.fori_loop(0, STEPS - 1, step_body, 0)
    _scatter_phase()
    plsc.subcore_barrier()
    _phase_e(True)


def _sc_call(xh, srcp, dstp):
    mesh = plsc.VectorSubcoreMesh(
        core_axis_name="c", subcore_axis_name="s",
        num_cores=NCORE, num_subcores=NTILE)
    fn = pl.kernel(
        _sc_body,
        out_type=[
            jax.ShapeDtypeStruct((NCORE, NP, DH), jnp.float32),
            jax.ShapeDtypeStruct((NCORE, NP, DH), jnp.float32),
        ],
        mesh=mesh,
        scratch_types=[
            pltpu.VMEM((NCH, CK), jnp.int32),
            pltpu.VMEM((NCH, CK), jnp.int32),
            pltpu.VMEM((2, CK, DH), jnp.float32),
            pltpu.VMEM((RBLK, DH), jnp.float32),
            pltpu.VMEM((RBLK, DH), jnp.float32),
            pltpu.VMEM((RBLK, DH), jnp.float32),
            pltpu.VMEM((RBLK, DH), jnp.float32),
            pltpu.VMEM((NP,), jnp.float32),
            pltpu.VMEM((RPT,), jnp.float32),
            pltpu.VMEM((RPT,), jnp.float32),
            pltpu.VMEM((RPT,), jnp.float32),
            pltpu.VMEM_SHARED((NTILE, NP), jnp.float32),
            pltpu.VMEM_SHARED((NP, DH), jnp.float32),
            pltpu.SemaphoreType.DMA,
        ],
    )
    return fn(xh, srcp, dstp)


def kernel(x, edge_index):
    src = edge_index[0].astype(jnp.int32)
    dst = edge_index[1].astype(jnp.int32)

    def prep(e):
        e = e.reshape(NTILE, EPT)
        pad = jnp.full((NTILE, EPT_PAD - EPT), NP - 1, jnp.int32)
        return jnp.concatenate([e, pad], axis=1).reshape(NTILE, NCH, CK)

    srcp = prep(src)
    dstp = prep(dst)
    xh = jnp.zeros((NCORE, NP, DH), jnp.float32)
    xh = xh.at[0, :N, :].set(x[:, :DH]).at[1, :N, :].set(x[:, DH:])
    yh, _ = _sc_call(xh, srcp, dstp)
    return jnp.concatenate([yh[0, :N], yh[1, :N]], axis=1)


# speedup vs baseline: 3.4965x; 3.4965x over previous
import jax
import jax.numpy as jnp
from jax import lax
from jax.experimental import pallas as pl
from jax.experimental.pallas import tpu as pltpu
from jax.experimental.pallas import tpu_sc as plsc

N = 10000
E = 320000
D = 128
DH = 64
STEPS = 5
NCORE = 2
NTILE = 16
NP = 10240
RPT = NP // NTILE
RBLK = 128
NRB = RPT // RBLK
EPT = E // NTILE
CK = 128
NCH = -(-EPT // CK)
EPT_PAD = NCH * CK
LANES = 16
FB = DH // LANES
GR = NP // DH
GRT = RPT // DH


def _sc_body(xh, srcp, dstp, y, hbuf,
             sidx, didx, rowbuf, xv, hv, aggv, zv,
             deg_v, degc_v, tmp_v, g_v, agg_sp, sem):
    c = lax.axis_index("c")
    t = lax.axis_index("s")
    rows_t = t * RPT
    f32 = jnp.float32
    ones16 = jnp.ones((LANES,), f32)
    zeros16 = jnp.zeros((LANES,), f32)

    def body_zero_z(i, carry):
        for f in range(FB):
            zv[i, pl.ds(f * LANES, LANES)] = zeros16
        return carry
    lax.fori_loop(0, RBLK, body_zero_z, 0)

    def body_zero_deg(i, carry):
        for f in range(FB):
            deg_v[i, pl.ds(f * LANES, LANES)] = zeros16
        return carry
    lax.fori_loop(0, GR, body_zero_deg, 0)

    def body_hist(j, carry):
        pltpu.sync_copy(dstp.at[t, j], didx.at[0])
        for k in range(CK // LANES):
            idx = didx[0, pl.ds(k * LANES, LANES)]
            plsc.addupdate_scatter(
                deg_v,
                [lax.shift_right_logical(idx, 6),
                 jnp.bitwise_and(idx, jnp.int32(DH - 1))],
                ones16)
        return carry
    lax.fori_loop(0, NCH, body_hist, 0)

    pltpu.sync_copy(deg_v, agg_sp.at[pl.ds(t * GR, GR)])
    plsc.subcore_barrier()

    pltpu.sync_copy(agg_sp.at[pl.ds(0 * GR + t * GRT, GRT)], degc_v)
    for u in range(1, NTILE):
        pltpu.sync_copy(agg_sp.at[pl.ds(u * GR + t * GRT, GRT)], tmp_v)

        def body_acc(k, carry):
            for f in range(FB):
                sl = pl.ds(f * LANES, LANES)
                degc_v[k, sl] = degc_v[k, sl] + tmp_v[k, sl]
            return carry
        lax.fori_loop(0, GRT, body_acc, 0)

    def body_g(k, carry):
        for f in range(FB):
            sl = pl.ds(f * LANES, LANES)
            d = degc_v[k, sl]
            i = plsc.bitcast(d, jnp.int32)
            i = jnp.int32(0x5F3759DF) - lax.shift_right_logical(i, 1)
            yb = plsc.bitcast(i, f32)
            hd = 0.5 * d
            for _ in range(4):
                yb = yb * (1.5 - hd * yb * yb)
            g_v[k, sl] = jnp.where(d > 0.5, yb, jnp.zeros_like(yb))
        return carry
    lax.fori_loop(0, GRT, body_g, 0)
    plsc.subcore_barrier()

    def _bg(r_local):
        hi = jnp.full((LANES,), lax.shift_right_logical(r_local, 6), jnp.int32)
        lo = jnp.full((LANES,), jnp.bitwise_and(r_local, DH - 1), jnp.int32)
        return plsc.load_gather(g_v, [hi, lo])

    for rb in range(NRB):
        rows0 = rows_t + rb * RBLK
        pltpu.sync_copy(xh.at[c, pl.ds(rows0, RBLK)], xv)

        def body_hi(r, carry, rb=rb):
            bg = _bg(rb * RBLK + r)
            for f in range(FB):
                sl = pl.ds(f * LANES, LANES)
                hv[r, sl] = bg * xv[r, sl]
            return carry
        lax.fori_loop(0, RBLK, body_hi, 0)
        pltpu.sync_copy(hv, hbuf.at[c, pl.ds(rows0, RBLK)])
        pltpu.sync_copy(zv, agg_sp.at[pl.ds(rows0, RBLK)])

    def _scatter_phase():
        def body(j, carry):
            pltpu.sync_copy(srcp.at[t, j], sidx.at[0])
            pltpu.sync_copy(dstp.at[t, j], didx.at[0])
            cp = pltpu.async_copy(hbuf.at[c].at[sidx.at[0]], rowbuf, sem)
            cp.wait()
            pltpu.sync_copy(rowbuf, agg_sp.at[didx.at[0]], add=True)
            return carry
        lax.fori_loop(0, NCH, body, 0)

    def _phase_e(final):
        for rb in range(NRB):
            rows0 = rows_t + rb * RBLK
            pltpu.sync_copy(agg_sp.at[pl.ds(rows0, RBLK)], aggv)
            if not final:
                pltpu.sync_copy(zv, agg_sp.at[pl.ds(rows0, RBLK)])
            pltpu.sync_copy(xh.at[c, pl.ds(rows0, RBLK)], xv)

            def body(r, carry, rb=rb):
                bg = _bg(rb * RBLK + r)
                for f in range(FB):
                    sl = pl.ds(f * LANES, LANES)
                    v = bg * aggv[r, sl] + xv[r, sl]
                    if final:
                        hv[r, sl] = 0.5 * v
                    else:
                        hv[r, sl] = (0.5 * bg) * v
                return carry
            lax.fori_loop(0, RBLK, body, 0)
            dstref = y if final else hbuf
            pltpu.sync_copy(hv, dstref.at[c, pl.ds(rows0, RBLK)])

    plsc.subcore_barrier()

    def step_body(k, carry):
        _scatter_phase()
        plsc.subcore_barrier()
        _phase_e(False)
        plsc.subcore_barrier()
        return carry
    lax.fori_loop(0, STEPS - 1, step_body, 0)
    _scatter_phase()
    plsc.subcore_barrier()
    _phase_e(True)


def _sc_call(xh, srcp, dstp):
    mesh = plsc.VectorSubcoreMesh(
        core_axis_name="c", subcore_axis_name="s",
        num_cores=NCORE, num_subcores=NTILE)
    fn = pl.kernel(
        _sc_body,
        out_type=[
            jax.ShapeDtypeStruct((NCORE, NP, DH), jnp.float32),
            jax.ShapeDtypeStruct((NCORE, NP, DH), jnp.float32),
        ],
        mesh=mesh,
        compiler_params=pltpu.CompilerParams(
            needs_layout_passes=False, use_tc_tiling_on_sc=False),
        scratch_types=[
            pltpu.VMEM((2, CK), jnp.int32),
            pltpu.VMEM((2, CK), jnp.int32),
            pltpu.VMEM((CK, DH), jnp.float32),
            pltpu.VMEM((RBLK, DH), jnp.float32),
            pltpu.VMEM((RBLK, DH), jnp.float32),
            pltpu.VMEM((RBLK, DH), jnp.float32),
            pltpu.VMEM((RBLK, DH), jnp.float32),
            pltpu.VMEM((GR, DH), jnp.float32),
            pltpu.VMEM((GRT, DH), jnp.float32),
            pltpu.VMEM((GRT, DH), jnp.float32),
            pltpu.VMEM((GRT, DH), jnp.float32),
            pltpu.VMEM_SHARED((NP, DH), jnp.float32),
            pltpu.SemaphoreType.DMA,
        ],
    )
    return fn(xh, srcp, dstp)


def kernel(x, edge_index):
    src = edge_index[0].astype(jnp.int32)
    dst = edge_index[1].astype(jnp.int32)

    def prep(e):
        e = e.reshape(NTILE, EPT)
        pad = jnp.full((NTILE, EPT_PAD - EPT), NP - 1, jnp.int32)
        return jnp.concatenate([e, pad], axis=1).reshape(NTILE, NCH, CK)

    srcp = prep(src)
    dstp = prep(dst)
    xh = jnp.zeros((NCORE, NP, DH), jnp.float32)
    xh = xh.at[0, :N, :].set(x[:, :DH]).at[1, :N, :].set(x[:, DH:])
    yh, _ = _sc_call(xh, srcp, dstp)
    return jnp.concatenate([yh[0, :N], yh[1, :N]], axis=1)


# speedup vs baseline: 4.9892x; 1.4269x over previous
import jax
import jax.numpy as jnp
from jax import lax
from jax.experimental import pallas as pl
from jax.experimental.pallas import tpu as pltpu
from jax.experimental.pallas import tpu_sc as plsc

N = 10000
E = 320000
D = 128
DH = 64
STEPS = 5
NCORE = 2
NTILE = 16
NP = 10240
RPT = NP // NTILE
RBLK = 128
NRB = RPT // RBLK
EPT = E // NTILE
CK = 128
NCH = 158
NCHA = NCH + 1
LANES = 16
FB = DH // LANES
ZR = 32
GR = NP // DH
GRT = RPT // DH


def _sc_body(xh, srcp, dstp, y, hbuf,
             src_v, dst_v, rowbuf, xv, aggv, zv,
             deg_v, degc_v, tmp_v, g_v, agg_sp, sem_g0, sem_g1):
    c = lax.axis_index("c")
    t = lax.axis_index("s")
    rows_t = t * RPT
    f32 = jnp.float32
    ones16 = jnp.ones((LANES,), f32)
    zeros16 = jnp.zeros((LANES,), f32)

    pltpu.sync_copy(srcp.at[t], src_v)
    pltpu.sync_copy(dstp.at[t], dst_v)

    def body_zero_z(i, carry):
        for f in range(FB):
            zv[i, pl.ds(f * LANES, LANES)] = zeros16
        return carry
    lax.fori_loop(0, ZR, body_zero_z, 0)

    def body_zero_deg(i, carry):
        for f in range(FB):
            deg_v[i, pl.ds(f * LANES, LANES)] = zeros16
        return carry
    lax.fori_loop(0, GR, body_zero_deg, 0)

    def body_hist(j, carry):
        for k in range(CK // LANES):
            idx = dst_v[j, pl.ds(k * LANES, LANES)]
            plsc.addupdate_scatter(
                deg_v,
                [lax.shift_right_logical(idx, 6),
                 jnp.bitwise_and(idx, jnp.int32(DH - 1))],
                ones16)
        return carry
    lax.fori_loop(0, NCH, body_hist, 0)

    pltpu.sync_copy(deg_v, agg_sp.at[pl.ds(t * GR, GR)])
    plsc.subcore_barrier()

    pltpu.sync_copy(agg_sp.at[pl.ds(t * GRT, GRT)], degc_v)
    for u in range(1, NTILE):
        pltpu.sync_copy(agg_sp.at[pl.ds(u * GR + t * GRT, GRT)], tmp_v)

        def body_acc(k, carry):
            for f in range(FB):
                sl = pl.ds(f * LANES, LANES)
                degc_v[k, sl] = degc_v[k, sl] + tmp_v[k, sl]
            return carry
        lax.fori_loop(0, GRT, body_acc, 0)

    def body_g(k, carry):
        for f in range(FB):
            sl = pl.ds(f * LANES, LANES)
            d = degc_v[k, sl]
            i = plsc.bitcast(d, jnp.int32)
            i = jnp.int32(0x5F3759DF) - lax.shift_right_logical(i, 1)
            yb = plsc.bitcast(i, f32)
            hd = 0.5 * d
            for _ in range(4):
                yb = yb * (1.5 - hd * yb * yb)
            g_v[k, sl] = jnp.where(d > 0.5, yb, jnp.zeros_like(yb))
        return carry
    lax.fori_loop(0, GRT, body_g, 0)
    plsc.subcore_barrier()

    def _bg(r_local):
        hi = jnp.full((LANES,), lax.shift_right_logical(r_local, 6), jnp.int32)
        lo = jnp.full((LANES,), jnp.bitwise_and(r_local, DH - 1), jnp.int32)
        return plsc.load_gather(g_v, [hi, lo])

    def _zero_agg_rows(rows0):
        for q in range(RBLK // ZR):
            pltpu.sync_copy(zv, agg_sp.at[pl.ds(rows0 + q * ZR, ZR)])

    for rb in range(NRB):
        rows0 = rows_t + rb * RBLK
        pltpu.sync_copy(xh.at[c, pl.ds(rows0, RBLK)], xv)

        def body_hi(r, carry, rb=rb):
            bg = _bg(rb * RBLK + r)
            for f in range(FB):
                sl = pl.ds(f * LANES, LANES)
                aggv[r, sl] = bg * xv[r, sl]
            return carry
        lax.fori_loop(0, RBLK, body_hi, 0)
        pltpu.sync_copy(aggv, hbuf.at[c, pl.ds(rows0, RBLK)])
        _zero_agg_rows(rows0)

    def _fire_gather(j, b, sem):
        return pltpu.async_copy(
            hbuf.at[c].at[src_v.at[j]], rowbuf.at[b], sem)

    def _wait_gather(b, sem):
        pltpu.make_async_copy(xh.at[0, pl.ds(0, CK)], rowbuf.at[b], sem).wait()

    def _scatter(j, b):
        pltpu.sync_copy(rowbuf.at[b], agg_sp.at[dst_v.at[j]], add=True)

    def _scatter_phase():
        _fire_gather(0, 0, sem_g0)

        def grp(q, carry):
            j0 = 2 * q
            _wait_gather(0, sem_g0)
            _fire_gather(j0 + 1, 1, sem_g1)
            _scatter(j0, 0)
            _wait_gather(1, sem_g1)
            _fire_gather(j0 + 2, 0, sem_g0)
            _scatter(j0 + 1, 1)
            return carry
        lax.fori_loop(0, NCH // 2, grp, 0)
        _wait_gather(0, sem_g0)

    def _phase_e(final):
        for rb in range(NRB):
            rows0 = rows_t + rb * RBLK
            pltpu.sync_copy(agg_sp.at[pl.ds(rows0, RBLK)], aggv)
            pltpu.sync_copy(xh.at[c, pl.ds(rows0, RBLK)], xv)
            if not final:
                _zero_agg_rows(rows0)

            def body(r, carry, rb=rb):
                bg = _bg(rb * RBLK + r)
                for f in range(FB):
                    sl = pl.ds(f * LANES, LANES)
                    v = bg * aggv[r, sl] + xv[r, sl]
                    if final:
                        aggv[r, sl] = 0.5 * v
                    else:
                        aggv[r, sl] = (0.5 * bg) * v
                return carry
            lax.fori_loop(0, RBLK, body, 0)
            dstref = y if final else hbuf
            pltpu.sync_copy(aggv, dstref.at[c, pl.ds(rows0, RBLK)])

    plsc.subcore_barrier()

    def step_body(k, carry):
        _scatter_phase()
        plsc.subcore_barrier()
        _phase_e(False)
        plsc.subcore_barrier()
        return carry
    lax.fori_loop(0, STEPS - 1, step_body, 0)
    _scatter_phase()
    plsc.subcore_barrier()
    _phase_e(True)


def _sc_call(xh, srcp, dstp):
    mesh = plsc.VectorSubcoreMesh(
        core_axis_name="c", subcore_axis_name="s",
        num_cores=NCORE, num_subcores=NTILE)
    fn = pl.kernel(
        _sc_body,
        out_type=[
            jax.ShapeDtypeStruct((NCORE, NP, DH), jnp.float32),
            jax.ShapeDtypeStruct((NCORE, NP, DH), jnp.float32),
        ],
        mesh=mesh,
        compiler_params=pltpu.CompilerParams(
            needs_layout_passes=False, use_tc_tiling_on_sc=False),
        scratch_types=[
            pltpu.VMEM((NCHA, CK), jnp.int32),
            pltpu.VMEM((NCHA, CK), jnp.int32),
            pltpu.VMEM((2, CK, DH), jnp.float32),
            pltpu.VMEM((RBLK, DH), jnp.float32),
            pltpu.VMEM((RBLK, DH), jnp.float32),
            pltpu.VMEM((ZR, DH), jnp.float32),
            pltpu.VMEM((GR, DH), jnp.float32),
            pltpu.VMEM((GRT, DH), jnp.float32),
            pltpu.VMEM((GRT, DH), jnp.float32),
            pltpu.VMEM((GRT, DH), jnp.float32),
            pltpu.VMEM_SHARED((NP, DH), jnp.float32),
            pltpu.SemaphoreType.DMA,
            pltpu.SemaphoreType.DMA,
        ],
    )
    return fn(xh, srcp, dstp)


def kernel(x, edge_index):
    src = edge_index[0].astype(jnp.int32)
    dst = edge_index[1].astype(jnp.int32)

    def prep(e):
        e = e.reshape(NTILE, EPT)
        pad = jnp.full((NTILE, NCHA * CK - EPT), NP - 1, jnp.int32)
        return jnp.concatenate([e, pad], axis=1).reshape(NTILE, NCHA, CK)

    srcp = prep(src)
    dstp = prep(dst)
    xh = jnp.zeros((NCORE, NP, DH), jnp.float32)
    xh = xh.at[0, :N, :].set(x[:, :DH]).at[1, :N, :].set(x[:, DH:])
    yh, _ = _sc_call(xh, srcp, dstp)
    return jnp.concatenate([yh[0, :N], yh[1, :N]], axis=1)
